# baseline (device time: 46436 ns/iter reference)
import contextlib
import os

import jax
import jax.numpy as jnp
from jax import lax
from jax.experimental import pallas as pl
from jax.experimental.pallas import tpu as pltpu

N_DEV = 8

_PROF = os.environ.get("K_PROF") == "1"


def _scope(name):
    return jax.named_scope(name) if _PROF else contextlib.nullcontext()

_LOG_MASK = {1: 1, 2: 3, 4: 4}
_RS_ORDERS = ((1, 2, 4), (2, 4, 1), (4, 1, 2))
_COLS = ((0, 384), (384, 384), (768, 256))


def kernel(x, W1, W2):
    m, k = x.shape
    _, h = W1.shape
    n = W2.shape[1]
    HC = 512
    f32 = jnp.float32
    bf16 = jnp.bfloat16

    def body(x_hbm, w1_hbm, w2_hbm, out_ref, xv_ref, x16_ref, w1v_ref, w2v_ref, h_ref,
             p0, p1, p2, ag0, ag1, ag2,
             s0a, s0b, s0c, s1a, s1b, s1c, s2a, s2b, s2c,
             r0a, r0b, r0c, r1a, r1b, r1c, r2a, r2b, r2c,
             send_sems, recv_sems, in_sems):
        p = (p0, p1, p2)
        ag = (ag0, ag1, ag2)
        sb = ((s0a, s0b, s0c), (s1a, s1b, s1c), (s2a, s2b, s2c))
        rb = ((r0a, r0b, r0c), (r1a, r1b, r1c), (r2a, r2b, r2c))

        my = lax.axis_index("i")
        vz = (my >> 2) & 1
        vy = (my >> 1) & 1
        vx = (my & 1) ^ vy
        vbit = {1: vx, 2: vy, 4: vz}
        partner = {mk: my ^ lm for mk, lm in _LOG_MASK.items()}
        bits = [[vbit[_RS_ORDERS[ki][r]] for r in range(3)] for ki in range(3)]
        prt = [[partner[_RS_ORDERS[ki][r]] for r in range(3)] for ki in range(3)]

        with _scope("w_dma_start"):
            cp_x = pltpu.make_async_copy(x_hbm, xv_ref, in_sems.at[h // HC + 1])
            cp_x.start()
            cp_w1 = [
                pltpu.make_async_copy(
                    w1_hbm.at[:, pl.ds(c * HC, HC)],
                    w1v_ref.at[:, pl.ds(c * HC, HC)],
                    in_sems.at[c],
                )
                for c in range(h // HC)
            ]
            for cp in cp_w1:
                cp.start()
            cp_w2 = pltpu.make_async_copy(w2_hbm, w2v_ref, in_sems.at[h // HC])
            cp_w2.start()

        with _scope("barrier"):
            barrier_sem = pltpu.get_barrier_semaphore()
            for mk in (1, 2, 4):
                pl.semaphore_signal(
                    barrier_sem, inc=1,
                    device_id=(partner[mk],), device_id_type=pl.DeviceIdType.MESH,
                )
            pl.semaphore_wait(barrier_sem, 3)

        with _scope("casts"):
            cp_x.wait()
            x16_ref[:, :] = xv_ref[:, :].astype(bf16)

        with _scope("gemm1"):
            for c in range(h // HC):
                cp_w1[c].wait()
                h_ref[:, c * HC:(c + 1) * HC] = jnp.maximum(
                    jnp.dot(x16_ref[:, :],
                            w1v_ref[:, c * HC:(c + 1) * HC].astype(bf16),
                            preferred_element_type=f32),
                    0.0,
                ).astype(bf16)

        half1 = m // 2
        halves = (m // 2, m // 4, m // 8)
        ss1 = [(1 - bits[ki][0]) * half1 for ki in range(3)]
        os1 = [bits[ki][0] * half1 for ki in range(3)]


        def start_rs(ki, r):
            rdma = pltpu.make_async_remote_copy(
                src_ref=sb[ki][r],
                dst_ref=rb[ki][r],
                send_sem=send_sems.at[ki, r],
                recv_sem=recv_sems.at[ki, r],
                device_id=(prt[ki][r],),
                device_id_type=pl.DeviceIdType.MESH,
            )
            rdma.start()
            return rdma

        with _scope("w2_dma_wait"):
            cp_w2.wait()

        rdmas = []
        for ki, (c0, w) in enumerate(_COLS):
            with _scope(f"gemm2_send_half{ki}"):
                sb[ki][0][:, :] = jnp.dot(
                    h_ref[pl.ds(ss1[ki], half1), :],
                    w2v_ref[:, c0:c0 + w].astype(bf16),
                    preferred_element_type=f32).astype(bf16)
            with _scope(f"rs1_start{ki}"):
                rdmas.append(start_rs(ki, 0))

        with _scope("gemm2_kept_half"):
            for ki, (c0, w) in enumerate(_COLS):
                p[ki][pl.ds(os1[ki], half1), :] = jnp.dot(
                    h_ref[pl.ds(os1[ki], half1), :],
                    w2v_ref[:, c0:c0 + w].astype(bf16),
                    preferred_element_type=f32)
        with _scope("rs1_wait"):
            for rdma in rdmas:
                rdma.wait()

        own = os1
        for r in (1, 2):
            half = halves[r]
            ss = [own[ki] + (1 - bits[ki][r]) * half for ki in range(3)]
            os_new = [own[ki] + bits[ki][r] * half for ki in range(3)]
            with _scope(f"rs{r+1}_sum_send"):
                for ki in range(3):
                    off = ss[ki] - own[ki]
                    sb[ki][r][:, :] = (
                        p[ki][pl.ds(ss[ki], half), :]
                        + rb[ki][r - 1][pl.ds(off, half), :].astype(f32)
                    ).astype(bf16)
            with _scope(f"rs{r+1}_start"):
                rdmas = [start_rs(ki, r) for ki in range(3)]
            with _scope(f"rs{r+1}_add_kept"):
                for ki in range(3):
                    off = os_new[ki] - own[ki]
                    p[ki][pl.ds(os_new[ki], half), :] = (
                        p[ki][pl.ds(os_new[ki], half), :]
                        + rb[ki][r - 1][pl.ds(off, half), :].astype(f32)
                    )
            with _scope(f"rs{r+1}_wait"):
                for rdma in rdmas:
                    rdma.wait()
            own = os_new

        def drain(ki, slot, cs, cl):
            c0, w = _COLS[ki]
            out_ref[pl.ds(cs, cl), c0:c0 + w] = p[ki][pl.ds(cs, cl), :]

        blk = m // N_DEV
        with _scope("rs3_final_add"):
            for ki, (c0, w) in enumerate(_COLS):
                red = (p[ki][pl.ds(own[ki], blk), :]
                       + rb[ki][2][:, :].astype(f32))
                p[ki][pl.ds(own[ki], blk), :] = red
                ag[ki][pl.ds(own[ki], blk), :] = red.astype(bf16)
                drain(ki, 0, own[ki], blk)

        for r in range(3):
            ln = blk << r
            rdmas = []
            new_own = []
            rstarts = []
            for ki in range(3):
                mk = _RS_ORDERS[ki][2 - r]
                b = vbit[mk]
                rdma = pltpu.make_async_remote_copy(
                    src_ref=ag[ki].at[pl.ds(own[ki], ln)],
                    dst_ref=ag[ki].at[pl.ds(own[ki], ln)],
                    send_sem=send_sems.at[ki, 3 + r],
                    recv_sem=recv_sems.at[ki, 3 + r],
                    device_id=(partner[mk],),
                    device_id_type=pl.DeviceIdType.MESH,
                )
                rdma.start()
                rdmas.append(rdma)
                ns = own[ki] - b * ln
                new_own.append(ns)
                rstarts.append(ns + (1 - b) * ln)
            if r > 0:
                with _scope(f"ag{r+1}_copy"):
                    for ki, (c0, w) in enumerate(_COLS):
                        cs, cl = prev_rstarts[ki], ln // 2
                        p[ki][pl.ds(cs, cl), :] = (
                            ag[ki][pl.ds(cs, cl), :].astype(f32)
                        )
                        drain(ki, r, cs, cl)
            with _scope(f"ag{r+1}_wait"):
                for rdma in rdmas:
                    rdma.wait()
            own = new_own
            prev_rstarts = rstarts

        with _scope("tail_copy"):
            for ki, (c0, w) in enumerate(_COLS):
                p[ki][pl.ds(prev_rstarts[ki], m // 2), :] = (
                    ag[ki][pl.ds(prev_rstarts[ki], m // 2), :].astype(f32)
                )
                drain(ki, 3, prev_rstarts[ki], m // 2)

    cols_w = [w for _, w in _COLS]
    stage_shapes = []
    for w in cols_w:
        stage_shapes += [
            pltpu.VMEM((m // 2, w), bf16),
            pltpu.VMEM((m // 4, w), bf16),
            pltpu.VMEM((m // 8, w), bf16),
        ]

    return pl.pallas_call(
        body,
        out_shape=jax.ShapeDtypeStruct((m, n), f32),
        in_specs=[
            pl.BlockSpec(memory_space=pl.ANY),
            pl.BlockSpec(memory_space=pl.ANY),
            pl.BlockSpec(memory_space=pl.ANY),
        ],
        out_specs=pl.BlockSpec(memory_space=pltpu.VMEM),
        scratch_shapes=[
            pltpu.VMEM((m, k), f32),
            pltpu.VMEM((m, k), bf16),
            pltpu.VMEM((k, h), f32),
            pltpu.VMEM((h, n), f32),
            pltpu.VMEM((m, h), bf16),
            pltpu.VMEM((m, cols_w[0]), f32),
            pltpu.VMEM((m, cols_w[1]), f32),
            pltpu.VMEM((m, cols_w[2]), f32),
            pltpu.VMEM((m, cols_w[0]), bf16),
            pltpu.VMEM((m, cols_w[1]), bf16),
            pltpu.VMEM((m, cols_w[2]), bf16),
            *stage_shapes,
            *stage_shapes,
            pltpu.SemaphoreType.DMA((3, 6)),
            pltpu.SemaphoreType.DMA((3, 6)),
            pltpu.SemaphoreType.DMA((6,)),
        ],
        compiler_params=pltpu.CompilerParams(collective_id=0, vmem_limit_bytes=100 * 1024 * 1024),
    )(x, W1, W2)


# device time: 45688 ns/iter; 1.0164x vs baseline; 1.0164x over previous
import contextlib
import os

import jax
import jax.numpy as jnp
from jax import lax
from jax.experimental import pallas as pl
from jax.experimental.pallas import tpu as pltpu

N_DEV = 8

_PROF = os.environ.get("K_PROF") == "1"


def _scope(name):
    return jax.named_scope(name) if _PROF else contextlib.nullcontext()

_LOG_MASK = {1: 1, 2: 3, 4: 4}
_RS_ORDERS = ((1, 2, 4), (2, 4, 1), (4, 1, 2))
_COLS = ((0, 384), (384, 384), (768, 256))


def kernel(x, W1, W2):
    m, k = x.shape
    _, h = W1.shape
    n = W2.shape[1]
    HC = 512
    f32 = jnp.float32
    bf16 = jnp.bfloat16

    def body(x_hbm, w1_hbm, w2_hbm, out_hbm, xv_ref, x16_ref, w1v_ref, w2v_ref, h_ref,
             p0, p1, p2, ag0, ag1, ag2,
             s0a, s0b, s0c, s1a, s1b, s1c, s2a, s2b, s2c,
             r0a, r0b, r0c, r1a, r1b, r1c, r2a, r2b, r2c,
             send_sems, recv_sems, in_sems, drain_sems):
        p = (p0, p1, p2)
        ag = (ag0, ag1, ag2)
        sb = ((s0a, s0b, s0c), (s1a, s1b, s1c), (s2a, s2b, s2c))
        rb = ((r0a, r0b, r0c), (r1a, r1b, r1c), (r2a, r2b, r2c))

        my = lax.axis_index("i")
        vz = (my >> 2) & 1
        vy = (my >> 1) & 1
        vx = (my & 1) ^ vy
        vbit = {1: vx, 2: vy, 4: vz}
        partner = {mk: my ^ lm for mk, lm in _LOG_MASK.items()}
        bits = [[vbit[_RS_ORDERS[ki][r]] for r in range(3)] for ki in range(3)]
        prt = [[partner[_RS_ORDERS[ki][r]] for r in range(3)] for ki in range(3)]

        with _scope("w_dma_start"):
            cp_x = pltpu.make_async_copy(x_hbm, xv_ref, in_sems.at[h // HC + 1])
            cp_x.start()
            cp_w1 = [
                pltpu.make_async_copy(
                    w1_hbm.at[:, pl.ds(c * HC, HC)],
                    w1v_ref.at[:, pl.ds(c * HC, HC)],
                    in_sems.at[c],
                )
                for c in range(h // HC)
            ]
            for cp in cp_w1:
                cp.start()
            cp_w2 = pltpu.make_async_copy(w2_hbm, w2v_ref, in_sems.at[h // HC])
            cp_w2.start()

        with _scope("barrier"):
            barrier_sem = pltpu.get_barrier_semaphore()
            for mk in (1, 2, 4):
                pl.semaphore_signal(
                    barrier_sem, inc=1,
                    device_id=(partner[mk],), device_id_type=pl.DeviceIdType.MESH,
                )
            pl.semaphore_wait(barrier_sem, 3)

        with _scope("casts"):
            cp_x.wait()
            x16_ref[:, :] = xv_ref[:, :].astype(bf16)

        with _scope("gemm1"):
            for c in range(h // HC):
                cp_w1[c].wait()
                h_ref[:, c * HC:(c + 1) * HC] = jnp.maximum(
                    jnp.dot(x16_ref[:, :],
                            w1v_ref[:, c * HC:(c + 1) * HC].astype(bf16),
                            preferred_element_type=f32),
                    0.0,
                ).astype(bf16)

        half1 = m // 2
        halves = (m // 2, m // 4, m // 8)
        ss1 = [(1 - bits[ki][0]) * half1 for ki in range(3)]
        os1 = [bits[ki][0] * half1 for ki in range(3)]


        def start_rs(ki, r):
            rdma = pltpu.make_async_remote_copy(
                src_ref=sb[ki][r],
                dst_ref=rb[ki][r],
                send_sem=send_sems.at[ki, r],
                recv_sem=recv_sems.at[ki, r],
                device_id=(prt[ki][r],),
                device_id_type=pl.DeviceIdType.MESH,
            )
            rdma.start()
            return rdma

        with _scope("w2_dma_wait"):
            cp_w2.wait()

        rdmas = []
        for ki, (c0, w) in enumerate(_COLS):
            with _scope(f"gemm2_send_half{ki}"):
                sb[ki][0][:, :] = jnp.dot(
                    h_ref[pl.ds(ss1[ki], half1), :],
                    w2v_ref[:, c0:c0 + w].astype(bf16),
                    preferred_element_type=f32).astype(bf16)
            with _scope(f"rs1_start{ki}"):
                rdmas.append(start_rs(ki, 0))

        with _scope("gemm2_kept_half"):
            for ki, (c0, w) in enumerate(_COLS):
                p[ki][pl.ds(os1[ki], half1), :] = jnp.dot(
                    h_ref[pl.ds(os1[ki], half1), :],
                    w2v_ref[:, c0:c0 + w].astype(bf16),
                    preferred_element_type=f32)
        with _scope("rs1_wait"):
            for rdma in rdmas:
                rdma.wait()

        own = os1
        for r in (1, 2):
            half = halves[r]
            ss = [own[ki] + (1 - bits[ki][r]) * half for ki in range(3)]
            os_new = [own[ki] + bits[ki][r] * half for ki in range(3)]
            with _scope(f"rs{r+1}_sum_send"):
                for ki in range(3):
                    off = ss[ki] - own[ki]
                    sb[ki][r][:, :] = (
                        p[ki][pl.ds(ss[ki], half), :]
                        + rb[ki][r - 1][pl.ds(off, half), :].astype(f32)
                    ).astype(bf16)
            with _scope(f"rs{r+1}_start"):
                rdmas = [start_rs(ki, r) for ki in range(3)]
            with _scope(f"rs{r+1}_add_kept"):
                for ki in range(3):
                    off = os_new[ki] - own[ki]
                    p[ki][pl.ds(os_new[ki], half), :] = (
                        p[ki][pl.ds(os_new[ki], half), :]
                        + rb[ki][r - 1][pl.ds(off, half), :].astype(f32)
                    )
            with _scope(f"rs{r+1}_wait"):
                for rdma in rdmas:
                    rdma.wait()
            own = os_new

        drains = []

        def drain(ki, slot, cs, cl):
            c0, w = _COLS[ki]
            dma = pltpu.make_async_copy(
                p[ki].at[pl.ds(cs, cl)],
                out_hbm.at[pl.ds(cs, cl), pl.ds(c0, w)],
                drain_sems.at[ki, slot],
            )
            dma.start()
            drains.append(dma)

        blk = m // N_DEV
        with _scope("rs3_final_add"):
            for ki, (c0, w) in enumerate(_COLS):
                red = (p[ki][pl.ds(own[ki], blk), :]
                       + rb[ki][2][:, :].astype(f32))
                p[ki][pl.ds(own[ki], blk), :] = red
                ag[ki][pl.ds(own[ki], blk), :] = red.astype(bf16)
                drain(ki, 0, own[ki], blk)

        for r in range(3):
            ln = blk << r
            rdmas = []
            new_own = []
            rstarts = []
            for ki in range(3):
                mk = _RS_ORDERS[ki][2 - r]
                b = vbit[mk]
                rdma = pltpu.make_async_remote_copy(
                    src_ref=ag[ki].at[pl.ds(own[ki], ln)],
                    dst_ref=ag[ki].at[pl.ds(own[ki], ln)],
                    send_sem=send_sems.at[ki, 3 + r],
                    recv_sem=recv_sems.at[ki, 3 + r],
                    device_id=(partner[mk],),
                    device_id_type=pl.DeviceIdType.MESH,
                )
                rdma.start()
                rdmas.append(rdma)
                ns = own[ki] - b * ln
                new_own.append(ns)
                rstarts.append(ns + (1 - b) * ln)
            if r > 0:
                with _scope(f"ag{r+1}_copy"):
                    for ki, (c0, w) in enumerate(_COLS):
                        cs, cl = prev_rstarts[ki], ln // 2
                        p[ki][pl.ds(cs, cl), :] = (
                            ag[ki][pl.ds(cs, cl), :].astype(f32)
                        )
                        drain(ki, r, cs, cl)
            with _scope(f"ag{r+1}_wait"):
                for rdma in rdmas:
                    rdma.wait()
            own = new_own
            prev_rstarts = rstarts

        with _scope("tail_copy"):
            for ki, (c0, w) in enumerate(_COLS):
                p[ki][pl.ds(prev_rstarts[ki], m // 2), :] = (
                    ag[ki][pl.ds(prev_rstarts[ki], m // 2), :].astype(f32)
                )
                drain(ki, 3, prev_rstarts[ki], m // 2)
        with _scope("drain_wait"):
            for dma in drains:
                dma.wait()

    cols_w = [w for _, w in _COLS]
    stage_shapes = []
    for w in cols_w:
        stage_shapes += [
            pltpu.VMEM((m // 2, w), bf16),
            pltpu.VMEM((m // 4, w), bf16),
            pltpu.VMEM((m // 8, w), bf16),
        ]

    return pl.pallas_call(
        body,
        out_shape=jax.ShapeDtypeStruct((m, n), f32),
        in_specs=[
            pl.BlockSpec(memory_space=pl.ANY),
            pl.BlockSpec(memory_space=pl.ANY),
            pl.BlockSpec(memory_space=pl.ANY),
        ],
        out_specs=pl.BlockSpec(memory_space=pl.ANY),
        scratch_shapes=[
            pltpu.VMEM((m, k), f32),
            pltpu.VMEM((m, k), bf16),
            pltpu.VMEM((k, h), f32),
            pltpu.VMEM((h, n), f32),
            pltpu.VMEM((m, h), bf16),
            pltpu.VMEM((m, cols_w[0]), f32),
            pltpu.VMEM((m, cols_w[1]), f32),
            pltpu.VMEM((m, cols_w[2]), f32),
            pltpu.VMEM((m, cols_w[0]), bf16),
            pltpu.VMEM((m, cols_w[1]), bf16),
            pltpu.VMEM((m, cols_w[2]), bf16),
            *stage_shapes,
            *stage_shapes,
            pltpu.SemaphoreType.DMA((3, 6)),
            pltpu.SemaphoreType.DMA((3, 6)),
            pltpu.SemaphoreType.DMA((6,)),
            pltpu.SemaphoreType.DMA((3, 4)),
        ],
        compiler_params=pltpu.CompilerParams(collective_id=0, vmem_limit_bytes=100 * 1024 * 1024),
    )(x, W1, W2)
